# baseline (speedup 1.0000x reference)
import functools

import jax
import jax.numpy as jnp
from jax import lax
from jax.experimental import pallas as pl
from jax.experimental.pallas import tpu as pltpu
from jax.experimental.pallas import tpu_sc as plsc

ALPHA = 0.1
BETA = 0.5 / (4 + 1)

NC = 2
NS = 16
NW = NC * NS


def _make_deg_kernel(E, N, chunk):
    epw = E // NW
    iters = epw // chunk
    mesh = plsc.VectorSubcoreMesh(core_axis_name="c", subcore_axis_name="s")

    @functools.partial(
        pl.kernel,
        out_type=jax.ShapeDtypeStruct((NC, 1, N), jnp.float32),
        mesh=mesh,
        scratch_types=[
            pltpu.VMEM((chunk,), jnp.int32),
            pltpu.VMEM((chunk,), jnp.float32),
            pltpu.VMEM_SHARED((N,), jnp.float32),
        ],
    )
    def deg_kernel(rows_hbm, ones_hbm, zeros_hbm, out_hbm, idx_v, ones_v, deg_sh):
        cid = lax.axis_index("c")
        sid = lax.axis_index("s")
        wid = sid * NC + cid

        @pl.when(sid == 0)
        def _():
            pltpu.sync_copy(zeros_hbm, deg_sh)

        pltpu.sync_copy(ones_hbm, ones_v)
        plsc.subcore_barrier()

        for i in range(iters):
            pltpu.sync_copy(rows_hbm.at[pl.ds(wid * epw + i * chunk, chunk)], idx_v)
            pltpu.sync_copy(ones_v, deg_sh.at[idx_v], add=True)

        plsc.subcore_barrier()

        @pl.when(sid == 0)
        def _():
            pltpu.sync_copy(deg_sh, out_hbm.at[cid, 0])

    return deg_kernel


def _make_spmm_kernel(epw_pad, N, D, chunk, npad, spad):
    iters = epw_pad // chunk
    rpt = npad // NS
    mesh = plsc.VectorSubcoreMesh(core_axis_name="c", subcore_axis_name="s")

    @functools.partial(
        pl.kernel,
        out_type=jax.ShapeDtypeStruct((NC, npad, D), jnp.float32),
        mesh=mesh,
        scratch_types=[
            pltpu.VMEM((chunk,), jnp.int32),
            pltpu.VMEM((chunk,), jnp.int32),
            pltpu.VMEM((chunk, D), jnp.float32),
            pltpu.VMEM_SHARED((spad, D), jnp.float32),
            pltpu.SemaphoreType.DMA,
        ],
    )
    def spmm_kernel(xs_hbm, cols_hbm, rows_hbm, zblk_hbm, out_hbm,
                    cidx_v, ridx_v, buf_v, s_sh, sem):
        cid = lax.axis_index("c")
        sid = lax.axis_index("s")
        wid = sid * NC + cid
        base = wid * epw_pad

        pltpu.sync_copy(zblk_hbm, s_sh.at[pl.ds(sid * rpt, rpt)])
        plsc.subcore_barrier()

        for i in range(iters):
            off = base + i * chunk
            pltpu.sync_copy(cols_hbm.at[pl.ds(off, chunk)], cidx_v)
            pltpu.sync_copy(rows_hbm.at[pl.ds(off, chunk)], ridx_v)
            pltpu.async_copy(xs_hbm.at[cidx_v], buf_v, sem).wait()
            pltpu.sync_copy(buf_v, s_sh.at[ridx_v], add=True)

        plsc.subcore_barrier()
        pltpu.sync_copy(s_sh.at[pl.ds(sid * rpt, rpt)],
                        out_hbm.at[cid, pl.ds(sid * rpt, rpt)])

    return spmm_kernel


def _scale_body(p0_ref, p1_ref, x_ref, xs_ref, dinv_ref):
    deg = 1.0 + p0_ref[...] + p1_ref[...]
    dinv = lax.rsqrt(deg)
    xs_ref[...] = x_ref[...] * dinv
    dinv_ref[...] = dinv


def _make_scale_kernel(N, D, blk):
    grid = N // blk
    return pl.pallas_call(
        _scale_body,
        grid=(grid,),
        in_specs=[
            pl.BlockSpec((blk, 1), lambda i: (i, 0)),
            pl.BlockSpec((blk, 1), lambda i: (i, 0)),
            pl.BlockSpec((blk, D), lambda i: (i, 0)),
        ],
        out_specs=[
            pl.BlockSpec((blk, D), lambda i: (i, 0)),
            pl.BlockSpec((blk, 1), lambda i: (i, 0)),
        ],
        out_shape=[
            jax.ShapeDtypeStruct((N, D), jnp.float32),
            jax.ShapeDtypeStruct((N, 1), jnp.float32),
        ],
    )


def _final_body(s_ref, xs_ref, x_ref, x0_ref, dinv_ref, w_ref, b_ref,
                out_ref):
    agg = dinv_ref[...] * (s_ref[0] + s_ref[1] + xs_ref[...])
    c1 = (1.0 - ALPHA) * BETA
    c2 = (1.0 - ALPHA) * (1.0 - BETA)
    t = c1 * agg + c2 * x_ref[...] + ALPHA * x0_ref[...]
    y = lax.dot_general(t, w_ref[...], (((1,), (1,)), ((), ())),
                        preferred_element_type=jnp.float32)
    out_ref[...] = y + b_ref[...]


def _make_final_kernel(N, D, blk, npad):
    grid = N // blk
    return pl.pallas_call(
        _final_body,
        grid=(grid,),
        in_specs=[
            pl.BlockSpec((NC, blk, D), lambda i: (0, i, 0)),
            pl.BlockSpec((blk, D), lambda i: (i, 0)),
            pl.BlockSpec((blk, D), lambda i: (i, 0)),
            pl.BlockSpec((blk, D), lambda i: (i, 0)),
            pl.BlockSpec((blk, 1), lambda i: (i, 0)),
            pl.BlockSpec((D, D), lambda i: (0, 0)),
            pl.BlockSpec((1, D), lambda i: (0, 0)),
        ],
        out_specs=pl.BlockSpec((blk, D), lambda i: (i, 0)),
        out_shape=jax.ShapeDtypeStruct((N, D), jnp.float32),
    )


DEG_CHUNK = 2000
SPMM_CHUNK = 344
TC_BLK = 2000


def _round_up(v, m):
    return (v + m - 1) // m * m


@jax.jit
def kernel(x, x0, edge_index, W, b):
    N, D = x.shape
    E = edge_index.shape[1]
    rows = edge_index[0]
    cols = edge_index[1]
    npad = _round_up(N, 8 * NS)

    ones_c = jnp.ones((DEG_CHUNK,), jnp.float32)
    zeros_n = jnp.zeros((N,), jnp.float32)
    zeros_blk = jnp.zeros((npad // NS, D), jnp.float32)

    deg_partial = _make_deg_kernel(E, N, DEG_CHUNK)(rows, ones_c, zeros_n)
    p0 = deg_partial[0, 0].reshape(N, 1)
    p1 = deg_partial[1, 0].reshape(N, 1)

    xs, dinv = _make_scale_kernel(N, D, TC_BLK)(p0, p1, x)

    epw = E // NW
    epw_pad = _round_up(epw, SPMM_CHUNK)
    spad = npad + 8
    if epw_pad != epw:
        fill = jnp.zeros((NW, epw_pad - epw), jnp.int32)
        cols_p = jnp.concatenate([cols.reshape(NW, epw), fill], axis=1).reshape(-1)
        rows_p = jnp.concatenate([rows.reshape(NW, epw), fill + npad],
                                 axis=1).reshape(-1)
    else:
        cols_p, rows_p = cols, rows
    s_partial = _make_spmm_kernel(epw_pad, N, D, SPMM_CHUNK, npad, spad)(
        xs, cols_p, rows_p, zeros_blk)

    out = _make_final_kernel(N, D, TC_BLK, npad)(
        s_partial, xs, x, x0, dinv, W, b.reshape(1, D))
    return out


# speedup vs baseline: 2.2958x; 2.2958x over previous
import functools

import jax
import jax.numpy as jnp
from jax import lax
from jax.experimental import pallas as pl
from jax.experimental.pallas import tpu as pltpu
from jax.experimental.pallas import tpu_sc as plsc

ALPHA = 0.1
BETA = 0.5 / (4 + 1)

NC = 2
NS = 16
NW = NC * NS


def _make_deg_kernel(E, N, chunk):
    epw = E // NW
    iters = epw // chunk
    mesh = plsc.VectorSubcoreMesh(core_axis_name="c", subcore_axis_name="s")

    @functools.partial(
        pl.kernel,
        out_type=jax.ShapeDtypeStruct((NC, 1, N), jnp.float32),
        mesh=mesh,
        scratch_types=[
            pltpu.VMEM((chunk,), jnp.int32),
            pltpu.VMEM((chunk,), jnp.float32),
            pltpu.VMEM_SHARED((N,), jnp.float32),
        ],
    )
    def deg_kernel(rows_hbm, ones_hbm, zeros_hbm, out_hbm, idx_v, ones_v, deg_sh):
        cid = lax.axis_index("c")
        sid = lax.axis_index("s")
        wid = sid * NC + cid

        @pl.when(sid == 0)
        def _():
            pltpu.sync_copy(zeros_hbm, deg_sh)

        pltpu.sync_copy(ones_hbm, ones_v)
        plsc.subcore_barrier()

        for i in range(iters):
            pltpu.sync_copy(rows_hbm.at[pl.ds(wid * epw + i * chunk, chunk)], idx_v)
            pltpu.sync_copy(ones_v, deg_sh.at[idx_v], add=True)

        plsc.subcore_barrier()

        @pl.when(sid == 0)
        def _():
            pltpu.sync_copy(deg_sh, out_hbm.at[cid, 0])

    return deg_kernel


def _make_spmm_kernel(epw_pad, N, D, chunk, npad, spad):
    iters = epw_pad // chunk
    rpt = npad // NS
    mesh = plsc.VectorSubcoreMesh(core_axis_name="c", subcore_axis_name="s")

    @functools.partial(
        pl.kernel,
        out_type=jax.ShapeDtypeStruct((NC, npad, D), jnp.float32),
        mesh=mesh,
        scratch_types=[
            pltpu.VMEM((chunk,), jnp.int32),
            pltpu.VMEM((chunk,), jnp.int32),
            pltpu.VMEM((chunk, D), jnp.float32),
            pltpu.VMEM_SHARED((spad, D), jnp.float32),
            pltpu.SemaphoreType.DMA,
        ],
    )
    def spmm_kernel(xs_hbm, cols_hbm, rows_hbm, zblk_hbm, out_hbm,
                    cidx_v, ridx_v, buf_v, s_sh, sem):
        cid = lax.axis_index("c")
        sid = lax.axis_index("s")
        wid = sid * NC + cid
        base = wid * epw_pad

        pltpu.sync_copy(zblk_hbm, s_sh.at[pl.ds(sid * rpt, rpt)])
        plsc.subcore_barrier()

        for i in range(iters):
            off = base + i * chunk
            pltpu.sync_copy(cols_hbm.at[pl.ds(off, chunk)], cidx_v)
            pltpu.sync_copy(rows_hbm.at[pl.ds(off, chunk)], ridx_v)
            pltpu.async_copy(xs_hbm.at[cidx_v], buf_v, sem).wait()
            pltpu.sync_copy(buf_v, s_sh.at[ridx_v], add=True)

        plsc.subcore_barrier()
        pltpu.sync_copy(s_sh.at[pl.ds(sid * rpt, rpt)],
                        out_hbm.at[cid, pl.ds(sid * rpt, rpt)])

    return spmm_kernel


def _scale_body(p0_ref, p1_ref, x_ref, xs_ref, dinv_ref):
    deg = 1.0 + p0_ref[...] + p1_ref[...]
    dinv = lax.rsqrt(deg)
    xs_ref[...] = x_ref[...] * dinv
    dinv_ref[...] = dinv


def _make_scale_kernel(N, D, blk):
    grid = N // blk
    return pl.pallas_call(
        _scale_body,
        grid=(grid,),
        in_specs=[
            pl.BlockSpec((blk, 1), lambda i: (i, 0)),
            pl.BlockSpec((blk, 1), lambda i: (i, 0)),
            pl.BlockSpec((blk, D), lambda i: (i, 0)),
        ],
        out_specs=[
            pl.BlockSpec((blk, D), lambda i: (i, 0)),
            pl.BlockSpec((blk, 1), lambda i: (i, 0)),
        ],
        out_shape=[
            jax.ShapeDtypeStruct((N, D), jnp.float32),
            jax.ShapeDtypeStruct((N, 1), jnp.float32),
        ],
    )


def _final_body(s_ref, xs_ref, x_ref, x0_ref, dinv_ref, w_ref, b_ref,
                out_ref):
    agg = dinv_ref[...] * (s_ref[0] + s_ref[1] + xs_ref[...])
    c1 = (1.0 - ALPHA) * BETA
    c2 = (1.0 - ALPHA) * (1.0 - BETA)
    t = c1 * agg + c2 * x_ref[...] + ALPHA * x0_ref[...]
    y = lax.dot_general(t, w_ref[...], (((1,), (1,)), ((), ())),
                        preferred_element_type=jnp.float32)
    out_ref[...] = y + b_ref[...]


def _make_final_kernel(N, D, blk, npad):
    grid = N // blk
    return pl.pallas_call(
        _final_body,
        grid=(grid,),
        in_specs=[
            pl.BlockSpec((NC, blk, D), lambda i: (0, i, 0)),
            pl.BlockSpec((blk, D), lambda i: (i, 0)),
            pl.BlockSpec((blk, D), lambda i: (i, 0)),
            pl.BlockSpec((blk, D), lambda i: (i, 0)),
            pl.BlockSpec((blk, 1), lambda i: (i, 0)),
            pl.BlockSpec((D, D), lambda i: (0, 0)),
            pl.BlockSpec((1, D), lambda i: (0, 0)),
        ],
        out_specs=pl.BlockSpec((blk, D), lambda i: (i, 0)),
        out_shape=jax.ShapeDtypeStruct((N, D), jnp.float32),
    )


DEG_CHUNK = 2000
SPMM_CHUNK = 200
TC_BLK = 2000


def _round_up(v, m):
    return (v + m - 1) // m * m


@jax.jit
def kernel(x, x0, edge_index, W, b):
    N, D = x.shape
    E = edge_index.shape[1]
    rows = edge_index[0]
    cols = edge_index[1]
    npad = _round_up(N, 8 * NS)

    ones_c = jnp.ones((DEG_CHUNK,), jnp.float32)
    zeros_n = jnp.zeros((N,), jnp.float32)
    zeros_blk = jnp.zeros((npad // NS, D), jnp.float32)

    deg_partial = _make_deg_kernel(E, N, DEG_CHUNK)(rows, ones_c, zeros_n)
    p0 = deg_partial[0, 0].reshape(N, 1)
    p1 = deg_partial[1, 0].reshape(N, 1)

    xs, dinv = _make_scale_kernel(N, D, TC_BLK)(p0, p1, x)

    epw = E // NW
    epw_pad = _round_up(epw, SPMM_CHUNK)
    spad = npad + 8
    if epw_pad != epw:
        fill = jnp.zeros((NW, epw_pad - epw), jnp.int32)
        cols_p = jnp.concatenate([cols.reshape(NW, epw), fill], axis=1).reshape(-1)
        rows_p = jnp.concatenate([rows.reshape(NW, epw), fill + npad],
                                 axis=1).reshape(-1)
    else:
        cols_p, rows_p = cols, rows
    s_partial = _make_spmm_kernel(epw_pad, N, D, SPMM_CHUNK, npad, spad)(
        xs, cols_p, rows_p, zeros_blk)

    out = _make_final_kernel(N, D, TC_BLK, npad)(
        s_partial, xs, x, x0, dinv, W, b.reshape(1, D))
    return out


# speedup vs baseline: 2.7159x; 1.1830x over previous
import functools

import jax
import jax.numpy as jnp
from jax import lax
from jax.experimental import pallas as pl
from jax.experimental.pallas import tpu as pltpu
from jax.experimental.pallas import tpu_sc as plsc

ALPHA = 0.1
BETA = 0.5 / (4 + 1)

NC = 2
NS = 16
NW = NC * NS


def _make_deg_kernel(E, N, chunk):
    epw = E // NW
    iters = epw // chunk
    mesh = plsc.VectorSubcoreMesh(core_axis_name="c", subcore_axis_name="s")

    @functools.partial(
        pl.kernel,
        out_type=jax.ShapeDtypeStruct((NC, 1, N), jnp.float32),
        mesh=mesh,
        scratch_types=[
            pltpu.VMEM((chunk,), jnp.int32),
            pltpu.VMEM((chunk,), jnp.float32),
            pltpu.VMEM_SHARED((N,), jnp.float32),
        ],
    )
    def deg_kernel(rows_hbm, ones_hbm, zeros_hbm, out_hbm, idx_v, ones_v, deg_sh):
        cid = lax.axis_index("c")
        sid = lax.axis_index("s")
        wid = sid * NC + cid

        @pl.when(sid == 0)
        def _():
            pltpu.sync_copy(zeros_hbm, deg_sh)

        pltpu.sync_copy(ones_hbm, ones_v)
        plsc.subcore_barrier()

        for i in range(iters):
            pltpu.sync_copy(rows_hbm.at[pl.ds(wid * epw + i * chunk, chunk)], idx_v)
            pltpu.sync_copy(ones_v, deg_sh.at[idx_v], add=True)

        plsc.subcore_barrier()

        @pl.when(sid == 0)
        def _():
            pltpu.sync_copy(deg_sh, out_hbm.at[cid, 0])

    return deg_kernel


def _make_spmm_kernel(epw_pad, N, D, chunk, npad, spad):
    iters = epw_pad // chunk
    rpt = npad // NS
    mesh = plsc.VectorSubcoreMesh(core_axis_name="c", subcore_axis_name="s")

    @functools.partial(
        pl.kernel,
        out_type=jax.ShapeDtypeStruct((NC, npad, D), jnp.float32),
        mesh=mesh,
        scratch_types=[
            pltpu.VMEM((chunk,), jnp.int32),
            pltpu.VMEM((chunk,), jnp.int32),
            pltpu.VMEM((chunk,), jnp.int32),
            pltpu.VMEM((chunk,), jnp.int32),
            pltpu.VMEM((chunk, D), jnp.float32),
            pltpu.VMEM_SHARED((spad, D), jnp.float32),
            pltpu.SemaphoreType.DMA,
            pltpu.SemaphoreType.DMA,
            pltpu.SemaphoreType.DMA,
        ],
    )
    def spmm_kernel(xs_hbm, cols_hbm, rows_hbm, zblk_hbm, out_hbm,
                    cidx0_v, cidx1_v, ridx0_v, ridx1_v, buf_v, s_sh,
                    gsem, isem0, isem1):
        cid = lax.axis_index("c")
        sid = lax.axis_index("s")
        wid = sid * NC + cid
        base = wid * epw_pad

        cidx = [cidx0_v, cidx1_v]
        ridx = [ridx0_v, ridx1_v]
        isems = [isem0, isem1]
        idescs = [None, None]

        pltpu.sync_copy(cols_hbm.at[pl.ds(base, chunk)], cidx[0])
        pltpu.sync_copy(rows_hbm.at[pl.ds(base, chunk)], ridx[0])
        if iters > 1:
            idescs[1] = [
                pltpu.async_copy(cols_hbm.at[pl.ds(base + chunk, chunk)],
                                 cidx[1], isems[1]),
                pltpu.async_copy(rows_hbm.at[pl.ds(base + chunk, chunk)],
                                 ridx[1], isems[1]),
            ]
        pltpu.sync_copy(zblk_hbm, s_sh.at[pl.ds(sid * rpt, rpt)])
        plsc.subcore_barrier()

        for i in range(iters):
            j = i & 1
            if idescs[j] is not None:
                for d in idescs[j]:
                    d.wait()
                idescs[j] = None
            pltpu.async_copy(xs_hbm.at[cidx[j]], buf_v, gsem).wait()
            pltpu.sync_copy(buf_v, s_sh.at[ridx[j]], add=True)
            if i + 2 < iters:
                off = base + (i + 2) * chunk
                idescs[j] = [
                    pltpu.async_copy(cols_hbm.at[pl.ds(off, chunk)], cidx[j],
                                     isems[j]),
                    pltpu.async_copy(rows_hbm.at[pl.ds(off, chunk)], ridx[j],
                                     isems[j]),
                ]

        plsc.subcore_barrier()
        pltpu.sync_copy(s_sh.at[pl.ds(sid * rpt, rpt)],
                        out_hbm.at[cid, pl.ds(sid * rpt, rpt)])

    return spmm_kernel


def _scale_body(p0_ref, p1_ref, x_ref, xs_ref, dinv_ref):
    deg = 1.0 + p0_ref[...] + p1_ref[...]
    dinv = lax.rsqrt(deg)
    xs_ref[...] = x_ref[...] * dinv
    dinv_ref[...] = dinv


def _make_scale_kernel(N, D, blk):
    grid = N // blk
    return pl.pallas_call(
        _scale_body,
        grid=(grid,),
        in_specs=[
            pl.BlockSpec((blk, 1), lambda i: (i, 0)),
            pl.BlockSpec((blk, 1), lambda i: (i, 0)),
            pl.BlockSpec((blk, D), lambda i: (i, 0)),
        ],
        out_specs=[
            pl.BlockSpec((blk, D), lambda i: (i, 0)),
            pl.BlockSpec((blk, 1), lambda i: (i, 0)),
        ],
        out_shape=[
            jax.ShapeDtypeStruct((N, D), jnp.float32),
            jax.ShapeDtypeStruct((N, 1), jnp.float32),
        ],
    )


def _final_body(s_ref, xs_ref, x_ref, x0_ref, dinv_ref, w_ref, b_ref,
                out_ref):
    agg = dinv_ref[...] * (s_ref[0] + s_ref[1] + xs_ref[...])
    c1 = (1.0 - ALPHA) * BETA
    c2 = (1.0 - ALPHA) * (1.0 - BETA)
    t = c1 * agg + c2 * x_ref[...] + ALPHA * x0_ref[...]
    y = lax.dot_general(t, w_ref[...], (((1,), (1,)), ((), ())),
                        preferred_element_type=jnp.float32)
    out_ref[...] = y + b_ref[...]


def _make_final_kernel(N, D, blk, npad):
    grid = N // blk
    return pl.pallas_call(
        _final_body,
        grid=(grid,),
        in_specs=[
            pl.BlockSpec((NC, blk, D), lambda i: (0, i, 0)),
            pl.BlockSpec((blk, D), lambda i: (i, 0)),
            pl.BlockSpec((blk, D), lambda i: (i, 0)),
            pl.BlockSpec((blk, D), lambda i: (i, 0)),
            pl.BlockSpec((blk, 1), lambda i: (i, 0)),
            pl.BlockSpec((D, D), lambda i: (0, 0)),
            pl.BlockSpec((1, D), lambda i: (0, 0)),
        ],
        out_specs=pl.BlockSpec((blk, D), lambda i: (i, 0)),
        out_shape=jax.ShapeDtypeStruct((N, D), jnp.float32),
    )


DEG_CHUNK = 2000
SPMM_CHUNK = 200
TC_BLK = 2000


def _round_up(v, m):
    return (v + m - 1) // m * m


@jax.jit
def kernel(x, x0, edge_index, W, b):
    N, D = x.shape
    E = edge_index.shape[1]
    rows = edge_index[0]
    cols = edge_index[1]
    npad = _round_up(N, 8 * NS)

    ones_c = jnp.ones((DEG_CHUNK,), jnp.float32)
    zeros_n = jnp.zeros((N,), jnp.float32)
    zeros_blk = jnp.zeros((npad // NS, D), jnp.float32)

    deg_partial = _make_deg_kernel(E, N, DEG_CHUNK)(rows, ones_c, zeros_n)
    p0 = deg_partial[0, 0].reshape(N, 1)
    p1 = deg_partial[1, 0].reshape(N, 1)

    xs, dinv = _make_scale_kernel(N, D, TC_BLK)(p0, p1, x)

    epw = E // NW
    epw_pad = _round_up(epw, SPMM_CHUNK)
    spad = npad + 8
    if epw_pad != epw:
        fill = jnp.zeros((NW, epw_pad - epw), jnp.int32)
        cols_p = jnp.concatenate([cols.reshape(NW, epw), fill], axis=1).reshape(-1)
        rows_p = jnp.concatenate([rows.reshape(NW, epw), fill + npad],
                                 axis=1).reshape(-1)
    else:
        cols_p, rows_p = cols, rows
    s_partial = _make_spmm_kernel(epw_pad, N, D, SPMM_CHUNK, npad, spad)(
        xs, cols_p, rows_p, zeros_blk)

    out = _make_final_kernel(N, D, TC_BLK, npad)(
        s_partial, xs, x, x0, dinv, W, b.reshape(1, D))
    return out


# speedup vs baseline: 3.2765x; 1.2064x over previous
import functools

import jax
import jax.numpy as jnp
from jax import lax
from jax.experimental import pallas as pl
from jax.experimental.pallas import tpu as pltpu
from jax.experimental.pallas import tpu_sc as plsc

ALPHA = 0.1
BETA = 0.5 / (4 + 1)

NC = 2
NS = 16
NW = NC * NS


def _make_deg_kernel(E, N, chunk):
    epw = E // NW
    iters = epw // chunk
    mesh = plsc.VectorSubcoreMesh(core_axis_name="c", subcore_axis_name="s")

    @functools.partial(
        pl.kernel,
        out_type=jax.ShapeDtypeStruct((NC, 1, N), jnp.float32),
        mesh=mesh,
        scratch_types=[
            pltpu.VMEM((chunk,), jnp.int32),
            pltpu.VMEM((chunk,), jnp.float32),
            pltpu.VMEM_SHARED((N,), jnp.float32),
        ],
    )
    def deg_kernel(rows_hbm, ones_hbm, zeros_hbm, out_hbm, idx_v, ones_v, deg_sh):
        cid = lax.axis_index("c")
        sid = lax.axis_index("s")
        wid = sid * NC + cid

        @pl.when(sid == 0)
        def _():
            pltpu.sync_copy(zeros_hbm, deg_sh)

        pltpu.sync_copy(ones_hbm, ones_v)
        plsc.subcore_barrier()

        for i in range(iters):
            pltpu.sync_copy(rows_hbm.at[pl.ds(wid * epw + i * chunk, chunk)], idx_v)
            pltpu.sync_copy(ones_v, deg_sh.at[idx_v], add=True)

        plsc.subcore_barrier()

        @pl.when(sid == 0)
        def _():
            pltpu.sync_copy(deg_sh, out_hbm.at[cid, 0])

    return deg_kernel


def _make_spmm_kernel(epw_pad, N, D, chunk, npad):
    iters = epw_pad // chunk
    rpt = npad // NS
    mesh = plsc.VectorSubcoreMesh(core_axis_name="c", subcore_axis_name="s")

    @functools.partial(
        pl.kernel,
        out_type=jax.ShapeDtypeStruct((NC, npad, D), jnp.float32),
        mesh=mesh,
        scratch_types=[
            pltpu.VMEM((chunk,), jnp.int32),
            pltpu.VMEM((chunk,), jnp.int32),
            pltpu.VMEM((chunk,), jnp.int32),
            pltpu.VMEM((chunk,), jnp.int32),
            pltpu.VMEM((chunk, D), jnp.float32),
            pltpu.VMEM((chunk, D), jnp.float32),
            pltpu.VMEM_SHARED((npad, D), jnp.float32),
            pltpu.SemaphoreType.DMA,
            pltpu.SemaphoreType.DMA,
            pltpu.SemaphoreType.DMA,
            pltpu.SemaphoreType.DMA,
        ],
    )
    def spmm_kernel(xs_hbm, cols_hbm, rows_hbm, zblk_hbm, out_hbm,
                    cidx0_v, cidx1_v, ridx0_v, ridx1_v, buf0_v, buf1_v, s_sh,
                    gsem0, gsem1, isem0, isem1):
        cid = lax.axis_index("c")
        sid = lax.axis_index("s")
        wid = sid * NC + cid
        base = wid * epw_pad

        cidx = [cidx0_v, cidx1_v]
        ridx = [ridx0_v, ridx1_v]
        isems = [isem0, isem1]
        idescs = [None, None]

        pltpu.sync_copy(cols_hbm.at[pl.ds(base, chunk)], cidx[0])
        pltpu.sync_copy(rows_hbm.at[pl.ds(base, chunk)], ridx[0])
        if iters > 1:
            idescs[1] = [
                pltpu.async_copy(cols_hbm.at[pl.ds(base + chunk, chunk)],
                                 cidx[1], isems[1]),
                pltpu.async_copy(rows_hbm.at[pl.ds(base + chunk, chunk)],
                                 ridx[1], isems[1]),
            ]
        pltpu.sync_copy(zblk_hbm, s_sh.at[pl.ds(sid * rpt, rpt)])
        plsc.subcore_barrier()

        bufs = [buf0_v, buf1_v]
        gsems = [gsem0, gsem1]
        gdescs = [None, None]
        gdescs[0] = pltpu.async_copy(xs_hbm.at[cidx[0]], bufs[0], gsems[0])
        for i in range(iters):
            j = i & 1
            if i + 1 < iters:
                if idescs[1 - j] is not None:
                    for d in idescs[1 - j]:
                        d.wait()
                    idescs[1 - j] = None
                gdescs[1 - j] = pltpu.async_copy(
                    xs_hbm.at[cidx[1 - j]], bufs[1 - j], gsems[1 - j])
            gdescs[j].wait()
            pltpu.sync_copy(bufs[j], s_sh.at[ridx[j]], add=True)
            if i + 2 < iters:
                off = base + (i + 2) * chunk
                idescs[j] = [
                    pltpu.async_copy(cols_hbm.at[pl.ds(off, chunk)], cidx[j],
                                     isems[j]),
                    pltpu.async_copy(rows_hbm.at[pl.ds(off, chunk)], ridx[j],
                                     isems[j]),
                ]

        plsc.subcore_barrier()
        pltpu.sync_copy(s_sh.at[pl.ds(sid * rpt, rpt)],
                        out_hbm.at[cid, pl.ds(sid * rpt, rpt)])

    return spmm_kernel


def _scale_body(p0_ref, p1_ref, x_ref, xs_ref, dinv_ref):
    deg = 1.0 + p0_ref[...] + p1_ref[...]
    dinv = lax.rsqrt(deg)
    xs_ref[...] = x_ref[...] * dinv
    dinv_ref[...] = dinv


def _make_scale_kernel(N, D, blk):
    grid = N // blk
    return pl.pallas_call(
        _scale_body,
        grid=(grid,),
        in_specs=[
            pl.BlockSpec((blk, 1), lambda i: (i, 0)),
            pl.BlockSpec((blk, 1), lambda i: (i, 0)),
            pl.BlockSpec((blk, D), lambda i: (i, 0)),
        ],
        out_specs=[
            pl.BlockSpec((blk, D), lambda i: (i, 0)),
            pl.BlockSpec((blk, 1), lambda i: (i, 0)),
        ],
        out_shape=[
            jax.ShapeDtypeStruct((N, D), jnp.float32),
            jax.ShapeDtypeStruct((N, 1), jnp.float32),
        ],
    )


def _final_body(s_ref, xs_ref, x_ref, x0_ref, dinv_ref, w_ref, b_ref,
                out_ref, *, blk, pad_cnt):
    s = s_ref[0] + s_ref[1] + xs_ref[...]
    if pad_cnt:
        grow = (pl.program_id(0) * blk
                + lax.broadcasted_iota(jnp.int32, (blk, 1), 0))
        s = s - jnp.where(grow < pad_cnt, float(NW), 0.0) * xs_ref[...]
    agg = dinv_ref[...] * s
    c1 = (1.0 - ALPHA) * BETA
    c2 = (1.0 - ALPHA) * (1.0 - BETA)
    t = c1 * agg + c2 * x_ref[...] + ALPHA * x0_ref[...]
    y = lax.dot_general(t, w_ref[...], (((1,), (1,)), ((), ())),
                        preferred_element_type=jnp.float32)
    out_ref[...] = y + b_ref[...]


def _make_final_kernel(N, D, blk, npad, pad_cnt):
    grid = N // blk
    return pl.pallas_call(
        functools.partial(_final_body, blk=blk, pad_cnt=pad_cnt),
        grid=(grid,),
        in_specs=[
            pl.BlockSpec((NC, blk, D), lambda i: (0, i, 0)),
            pl.BlockSpec((blk, D), lambda i: (i, 0)),
            pl.BlockSpec((blk, D), lambda i: (i, 0)),
            pl.BlockSpec((blk, D), lambda i: (i, 0)),
            pl.BlockSpec((blk, 1), lambda i: (i, 0)),
            pl.BlockSpec((D, D), lambda i: (0, 0)),
            pl.BlockSpec((1, D), lambda i: (0, 0)),
        ],
        out_specs=pl.BlockSpec((blk, D), lambda i: (i, 0)),
        out_shape=jax.ShapeDtypeStruct((N, D), jnp.float32),
    )


DEG_CHUNK = 2000
SPMM_CHUNK = 192
TC_BLK = 2000


def _round_up(v, m):
    return (v + m - 1) // m * m


@jax.jit
def kernel(x, x0, edge_index, W, b):
    N, D = x.shape
    E = edge_index.shape[1]
    rows = edge_index[0]
    cols = edge_index[1]
    npad = _round_up(N, 8 * NS)

    ones_c = jnp.ones((DEG_CHUNK,), jnp.float32)
    zeros_n = jnp.zeros((N,), jnp.float32)
    zeros_blk = jnp.zeros((npad // NS, D), jnp.float32)

    deg_partial = _make_deg_kernel(E, N, DEG_CHUNK)(rows, ones_c, zeros_n)
    p0 = deg_partial[0, 0].reshape(N, 1)
    p1 = deg_partial[1, 0].reshape(N, 1)

    xs, dinv = _make_scale_kernel(N, D, TC_BLK)(p0, p1, x)

    epw = E // NW
    epw_pad = _round_up(epw, SPMM_CHUNK)
    pad_cnt = epw_pad - epw
    if pad_cnt:
        fill = jnp.broadcast_to(jnp.arange(pad_cnt, dtype=jnp.int32),
                                (NW, pad_cnt))
        cols_p = jnp.concatenate([cols.reshape(NW, epw), fill],
                                 axis=1).reshape(-1)
        rows_p = jnp.concatenate([rows.reshape(NW, epw), fill],
                                 axis=1).reshape(-1)
    else:
        cols_p, rows_p = cols, rows
    s_partial = _make_spmm_kernel(epw_pad, N, D, SPMM_CHUNK, npad)(
        xs, cols_p, rows_p, zeros_blk)

    out = _make_final_kernel(N, D, TC_BLK, npad, pad_cnt)(
        s_partial, xs, x, x0, dinv, W, b.reshape(1, D))
    return out


# speedup vs baseline: 3.2956x; 1.0059x over previous
import functools

import jax
import jax.numpy as jnp
from jax import lax
from jax.experimental import pallas as pl
from jax.experimental.pallas import tpu as pltpu
from jax.experimental.pallas import tpu_sc as plsc

ALPHA = 0.1
BETA = 0.5 / (4 + 1)

NC = 2
NS = 16
NW = NC * NS


def _make_deg_kernel(E, N, chunk):
    epw = E // NW
    iters = epw // chunk
    mesh = plsc.VectorSubcoreMesh(core_axis_name="c", subcore_axis_name="s")

    @functools.partial(
        pl.kernel,
        out_type=jax.ShapeDtypeStruct((NC, 1, N), jnp.float32),
        mesh=mesh,
        scratch_types=[
            pltpu.VMEM((chunk,), jnp.int32),
            pltpu.VMEM((chunk,), jnp.float32),
            pltpu.VMEM_SHARED((N,), jnp.float32),
        ],
    )
    def deg_kernel(rows_hbm, ones_hbm, zeros_hbm, out_hbm, idx_v, ones_v, deg_sh):
        cid = lax.axis_index("c")
        sid = lax.axis_index("s")
        wid = sid * NC + cid

        @pl.when(sid == 0)
        def _():
            pltpu.sync_copy(zeros_hbm, deg_sh)

        pltpu.sync_copy(ones_hbm, ones_v)
        plsc.subcore_barrier()

        for i in range(iters):
            pltpu.sync_copy(rows_hbm.at[pl.ds(wid * epw + i * chunk, chunk)], idx_v)
            pltpu.sync_copy(ones_v, deg_sh.at[idx_v], add=True)

        plsc.subcore_barrier()

        @pl.when(sid == 0)
        def _():
            pltpu.sync_copy(deg_sh, out_hbm.at[cid, 0])

    return deg_kernel


def _make_spmm_kernel(epw_pad, N, D, chunk, npad):
    iters = epw_pad // chunk
    rpt = npad // NS
    mesh = plsc.VectorSubcoreMesh(core_axis_name="c", subcore_axis_name="s")

    @functools.partial(
        pl.kernel,
        out_type=jax.ShapeDtypeStruct((NC, npad, D), jnp.float32),
        mesh=mesh,
        scratch_types=[
            pltpu.VMEM((chunk,), jnp.int32),
            pltpu.VMEM((chunk,), jnp.int32),
            pltpu.VMEM((chunk,), jnp.int32),
            pltpu.VMEM((chunk,), jnp.int32),
            pltpu.VMEM((chunk, D), jnp.float32),
            pltpu.VMEM((chunk, D), jnp.float32),
            pltpu.VMEM_SHARED((npad, D), jnp.float32),
            pltpu.SemaphoreType.DMA,
            pltpu.SemaphoreType.DMA,
            pltpu.SemaphoreType.DMA,
            pltpu.SemaphoreType.DMA,
        ],
    )
    def spmm_kernel(xs_hbm, cols_hbm, rows_hbm, zblk_hbm, out_hbm,
                    cidx0_v, cidx1_v, ridx0_v, ridx1_v, buf0_v, buf1_v, s_sh,
                    gsem0, gsem1, isem0, isem1):
        cid = lax.axis_index("c")
        sid = lax.axis_index("s")
        wid = sid * NC + cid
        base = wid * epw_pad

        cidx = [cidx0_v, cidx1_v]
        ridx = [ridx0_v, ridx1_v]
        isems = [isem0, isem1]
        idescs = [None, None]

        pltpu.sync_copy(cols_hbm.at[pl.ds(base, chunk)], cidx[0])
        pltpu.sync_copy(rows_hbm.at[pl.ds(base, chunk)], ridx[0])
        if iters > 1:
            idescs[1] = [
                pltpu.async_copy(cols_hbm.at[pl.ds(base + chunk, chunk)],
                                 cidx[1], isems[1]),
                pltpu.async_copy(rows_hbm.at[pl.ds(base + chunk, chunk)],
                                 ridx[1], isems[1]),
            ]
        pltpu.sync_copy(zblk_hbm, s_sh.at[pl.ds(sid * rpt, rpt)])
        plsc.subcore_barrier()

        bufs = [buf0_v, buf1_v]
        gsems = [gsem0, gsem1]
        gdescs = [None, None]
        gdescs[0] = pltpu.async_copy(xs_hbm.at[cidx[0]], bufs[0], gsems[0])
        for i in range(iters):
            j = i & 1
            if i + 1 < iters:
                if idescs[1 - j] is not None:
                    for d in idescs[1 - j]:
                        d.wait()
                    idescs[1 - j] = None
                gdescs[1 - j] = pltpu.async_copy(
                    xs_hbm.at[cidx[1 - j]], bufs[1 - j], gsems[1 - j])
            gdescs[j].wait()
            pltpu.sync_copy(bufs[j], s_sh.at[ridx[j]], add=True)
            if i + 2 < iters:
                off = base + (i + 2) * chunk
                idescs[j] = [
                    pltpu.async_copy(cols_hbm.at[pl.ds(off, chunk)], cidx[j],
                                     isems[j]),
                    pltpu.async_copy(rows_hbm.at[pl.ds(off, chunk)], ridx[j],
                                     isems[j]),
                ]

        plsc.subcore_barrier()
        pltpu.sync_copy(s_sh.at[pl.ds(sid * rpt, rpt)],
                        out_hbm.at[cid, pl.ds(sid * rpt, rpt)])

    return spmm_kernel


def _scale_body(p0_ref, p1_ref, x_ref, xs_ref, dinv_ref):
    deg = 1.0 + p0_ref[...] + p1_ref[...]
    dinv = lax.rsqrt(deg)
    xs_ref[...] = x_ref[...] * dinv
    dinv_ref[...] = dinv


def _make_scale_kernel(N, D, blk):
    grid = N // blk
    return pl.pallas_call(
        _scale_body,
        grid=(grid,),
        in_specs=[
            pl.BlockSpec((blk, 1), lambda i: (i, 0)),
            pl.BlockSpec((blk, 1), lambda i: (i, 0)),
            pl.BlockSpec((blk, D), lambda i: (i, 0)),
        ],
        out_specs=[
            pl.BlockSpec((blk, D), lambda i: (i, 0)),
            pl.BlockSpec((blk, 1), lambda i: (i, 0)),
        ],
        out_shape=[
            jax.ShapeDtypeStruct((N, D), jnp.float32),
            jax.ShapeDtypeStruct((N, 1), jnp.float32),
        ],
    )


def _final_body(s_ref, xs_ref, x_ref, x0_ref, dinv_ref, w_ref, b_ref,
                out_ref, *, blk, pad_cnt):
    s = s_ref[0] + s_ref[1] + xs_ref[...]
    if pad_cnt:
        grow = (pl.program_id(0) * blk
                + lax.broadcasted_iota(jnp.int32, (blk, 1), 0))
        s = s - jnp.where(grow < pad_cnt, float(NW), 0.0) * xs_ref[...]
    agg = dinv_ref[...] * s
    c1 = (1.0 - ALPHA) * BETA
    c2 = (1.0 - ALPHA) * (1.0 - BETA)
    t = c1 * agg + c2 * x_ref[...] + ALPHA * x0_ref[...]
    y = lax.dot_general(t, w_ref[...], (((1,), (1,)), ((), ())),
                        preferred_element_type=jnp.float32)
    out_ref[...] = y + b_ref[...]


def _make_final_kernel(N, D, blk, npad, pad_cnt):
    grid = N // blk
    return pl.pallas_call(
        functools.partial(_final_body, blk=blk, pad_cnt=pad_cnt),
        grid=(grid,),
        in_specs=[
            pl.BlockSpec((NC, blk, D), lambda i: (0, i, 0)),
            pl.BlockSpec((blk, D), lambda i: (i, 0)),
            pl.BlockSpec((blk, D), lambda i: (i, 0)),
            pl.BlockSpec((blk, D), lambda i: (i, 0)),
            pl.BlockSpec((blk, 1), lambda i: (i, 0)),
            pl.BlockSpec((D, D), lambda i: (0, 0)),
            pl.BlockSpec((1, D), lambda i: (0, 0)),
        ],
        out_specs=pl.BlockSpec((blk, D), lambda i: (i, 0)),
        out_shape=jax.ShapeDtypeStruct((N, D), jnp.float32),
    )


DEG_CHUNK = 10000
SPMM_CHUNK = 192
TC_BLK = 2000


def _round_up(v, m):
    return (v + m - 1) // m * m


@jax.jit
def kernel(x, x0, edge_index, W, b):
    N, D = x.shape
    E = edge_index.shape[1]
    rows = edge_index[0]
    cols = edge_index[1]
    npad = _round_up(N, 8 * NS)

    ones_c = jnp.ones((DEG_CHUNK,), jnp.float32)
    zeros_n = jnp.zeros((N,), jnp.float32)
    zeros_blk = jnp.zeros((npad // NS, D), jnp.float32)

    deg_partial = _make_deg_kernel(E, N, DEG_CHUNK)(rows, ones_c, zeros_n)
    p0 = deg_partial[0, 0].reshape(N, 1)
    p1 = deg_partial[1, 0].reshape(N, 1)

    xs, dinv = _make_scale_kernel(N, D, TC_BLK)(p0, p1, x)

    epw = E // NW
    epw_pad = _round_up(epw, SPMM_CHUNK)
    pad_cnt = epw_pad - epw
    if pad_cnt:
        fill = jnp.broadcast_to(jnp.arange(pad_cnt, dtype=jnp.int32),
                                (NW, pad_cnt))
        cols_p = jnp.concatenate([cols.reshape(NW, epw), fill],
                                 axis=1).reshape(-1)
        rows_p = jnp.concatenate([rows.reshape(NW, epw), fill],
                                 axis=1).reshape(-1)
    else:
        cols_p, rows_p = cols, rows
    s_partial = _make_spmm_kernel(epw_pad, N, D, SPMM_CHUNK, npad)(
        xs, cols_p, rows_p, zeros_blk)

    out = _make_final_kernel(N, D, TC_BLK, npad, pad_cnt)(
        s_partial, xs, x, x0, dinv, W, b.reshape(1, D))
    return out
